# baseline (speedup 1.0000x reference)
import jax
import jax.numpy as jnp
from jax import lax
from jax.experimental import pallas as pl
from jax.experimental.pallas import tpu as pltpu
from jax.experimental.pallas import tpu_sc as plsc

_B, _Q, _C = 64, 2048, 256
_R = _B * _Q
_NC, _NS, _L = 2, 16, 16
_NW = _NC * _NS
_RPW = _R // _NW
_CHUNK = 128
_NCHUNK = _RPW // _CHUNK
_GROUPS = _CHUNK // _L


def _sc_body(logits_hbm, boxes_hbm, labels_hbm, boxes_out_hbm, counts_hbm,
             lbuf0, lbuf1, bbuf, lab_st, box_st, vscr, sem_b, sem0, sem1):
    cid = lax.axis_index("c")
    sid = lax.axis_index("s")
    wid = sid * _NC + cid
    rbase = wid * _RPW

    lane = lax.iota(jnp.int32, _L)
    lane_c = lane * _C
    lane_d4 = jnp.right_shift(lane, 2)
    box_sel = [lane_d4 + (4 * k) for k in range(4)]

    lbufs = (lbuf0, lbuf1)
    sems = (sem0, sem1)

    boxes_cp = pltpu.async_copy(
        boxes_hbm.at[pl.ds(rbase * 4, _RPW * 4)], bbuf, sem_b)
    pltpu.async_copy(
        logits_hbm.at[pl.ds(rbase * _C, _CHUNK * _C)], lbuf0, sem0)
    pltpu.async_copy(
        logits_hbm.at[pl.ds((rbase + _CHUNK) * _C, _CHUNK * _C)], lbuf1, sem1)
    boxes_cp.wait()

    def chunk_step(g, b, cnt):
        buf = lbufs[b]
        sem = sems[b]
        pltpu.make_async_copy(
            logits_hbm.at[pl.ds(rbase * _C, _CHUNK * _C)], buf, sem).wait()

        def grp_body(grp, cnt):
            row0 = g * _CHUNK + grp * _L
            idx0 = lane_c + grp * (_L * _C)
            best = plsc.load_gather(buf, [idx0])

            def cbody(_, carry):
                best, bidx, idx = carry
                idx = idx + 1
                v = plsc.load_gather(buf, [idx])
                gt = v > best
                best = jnp.where(gt, v, best)
                bidx = jnp.where(gt, idx, bidx)
                return (best, bidx, idx)

            best, bidx, _ = lax.fori_loop(
                1, _C, cbody, (best, idx0, idx0), unroll=8)
            cls = bidx - idx0
            valid = best > 0.0
            lab_st[pl.ds(row0, _L)] = jnp.where(valid, cls, -1)
            cnt = cnt + jnp.where(valid, 1.0, 0.0)
            vscr[...] = jnp.where(valid, 1.0, 0.0)
            for k in range(4):
                mv = plsc.load_gather(vscr, [box_sel[k]])
                off = row0 * 4 + k * _L
                box_st[pl.ds(off, _L)] = jnp.where(
                    mv > 0.0, bbuf[pl.ds(off, _L)], 0.0)
            return cnt

        cnt = lax.fori_loop(0, _GROUPS, grp_body, cnt)

        @pl.when(g + 2 < _NCHUNK)
        def _():
            pltpu.async_copy(
                logits_hbm.at[pl.ds((rbase + (g + 2) * _CHUNK) * _C,
                                    _CHUNK * _C)], buf, sem)

        return cnt

    def pair_body(p, cnt):
        g = p * 2
        cnt = chunk_step(g, 0, cnt)
        cnt = chunk_step(g + 1, 1, cnt)
        return cnt

    cnt = lax.fori_loop(0, _NCHUNK // 2, pair_body,
                        jnp.zeros((_L,), jnp.float32))

    vscr[...] = cnt
    pltpu.sync_copy(vscr, counts_hbm.at[wid])
    pltpu.sync_copy(lab_st, labels_hbm.at[pl.ds(rbase, _RPW)])
    pltpu.sync_copy(box_st, boxes_out_hbm.at[pl.ds(rbase * 4, _RPW * 4)])


def _finalize_body(cref, oref):
    oref[...] = lax.broadcast(jnp.maximum(jnp.sum(cref[...]), 1.0), (1, 1))


def kernel(pred_logits, pred_boxes):
    logits_flat = pred_logits.reshape(_R * _C)
    boxes_flat = pred_boxes.reshape(_R * 4)
    mesh = plsc.VectorSubcoreMesh(core_axis_name="c", subcore_axis_name="s")
    labels_flat, boxes_out, counts = pl.kernel(
        _sc_body,
        out_type=(
            jax.ShapeDtypeStruct((_R,), jnp.int32),
            jax.ShapeDtypeStruct((_R * 4,), jnp.float32),
            jax.ShapeDtypeStruct((_NW, _L), jnp.float32),
        ),
        mesh=mesh,
        compiler_params=pltpu.CompilerParams(needs_layout_passes=False),
        scratch_types=[
            pltpu.VMEM((_CHUNK * _C,), jnp.float32),
            pltpu.VMEM((_CHUNK * _C,), jnp.float32),
            pltpu.VMEM((_RPW * 4,), jnp.float32),
            pltpu.VMEM((_RPW,), jnp.int32),
            pltpu.VMEM((_RPW * 4,), jnp.float32),
            pltpu.VMEM((_L,), jnp.float32),
            pltpu.SemaphoreType.DMA,
            pltpu.SemaphoreType.DMA,
            pltpu.SemaphoreType.DMA,
        ],
    )(logits_flat, boxes_flat)
    num_boxes = pl.pallas_call(
        _finalize_body,
        out_shape=jax.ShapeDtypeStruct((1, 1), jnp.float32),
    )(counts)[0, 0]
    return (labels_flat.reshape(_B, _Q),
            boxes_out.reshape(_B, _Q, 4),
            num_boxes)


# speedup vs baseline: 2.0613x; 2.0613x over previous
import jax
import jax.numpy as jnp
from jax import lax
from jax.experimental import pallas as pl
from jax.experimental.pallas import tpu as pltpu
from jax.experimental.pallas import tpu_sc as plsc

_B, _Q, _C = 64, 2048, 256
_R = _B * _Q
_NC, _NS, _L = 2, 16, 16
_NW = _NC * _NS
_RPW = _R // _NW
_CHUNK = 128
_NCHUNK = _RPW // _CHUNK
_GROUPS = _CHUNK // _L


def _sc_body(logits_hbm, boxes_hbm, labels_hbm, boxes_out_hbm, counts_hbm,
             lbuf0, lbuf1, bbuf, lab_st, box_st, vscr, sem_b, sem0, sem1):
    cid = lax.axis_index("c")
    sid = lax.axis_index("s")
    wid = sid * _NC + cid
    rbase = wid * _RPW
    b0 = wid * (_RPW // _Q)

    lane = lax.iota(jnp.int32, _L)
    lane_d4 = jnp.right_shift(lane, 2)
    box_sel = [lane_d4 + (4 * k) for k in range(4)]

    lbufs = (lbuf0, lbuf1)
    sems = (sem0, sem1)

    def start_chunk_dma(g, buf, sem):
        bb = b0 + jnp.right_shift(g, 4)
        q0 = jnp.bitwise_and(g, 15) * _CHUNK
        pltpu.async_copy(logits_hbm.at[bb, pl.ds(q0, _CHUNK), :], buf, sem)

    boxes_cp = pltpu.async_copy(
        boxes_hbm.at[pl.ds(rbase * 4, _RPW * 4)], bbuf, sem_b)
    start_chunk_dma(jnp.int32(0), lbuf0, sem0)
    start_chunk_dma(jnp.int32(1), lbuf1, sem1)
    boxes_cp.wait()

    def chunk_step(g, b, cnt):
        buf = lbufs[b]
        sem = sems[b]
        pltpu.make_async_copy(
            logits_hbm.at[0, pl.ds(0, _CHUNK), :], buf, sem).wait()

        def grp_body(grp, cnt):
            row0 = g * _CHUNK + grp * _L
            rows = lane + grp * _L
            best = plsc.load_gather(buf, [rows, lane])

            def cbody(_, carry):
                best, bcol, col = carry
                col = col + 1
                v = plsc.load_gather(buf, [rows, col])
                gt = v > best
                best = jnp.where(gt, v, best)
                bcol = jnp.where(gt, col, bcol)
                return (best, bcol, col)

            best, bcol, col = lax.fori_loop(
                1, 240, cbody, (best, lane, lane), unroll=8)

            def cbody_wrap(_, carry):
                best, bcol, col = carry
                col = jnp.bitwise_and(col + 1, 255)
                v = plsc.load_gather(buf, [rows, col])
                gt = v > best
                best = jnp.where(gt, v, best)
                bcol = jnp.where(gt, col, bcol)
                return (best, bcol, col)

            best, bcol, _ = lax.fori_loop(
                240, 256, cbody_wrap, (best, bcol, col), unroll=8)
            cls = bcol
            valid = best > 0.0
            lab_st[pl.ds(row0, _L)] = jnp.where(valid, cls, -1)
            cnt = cnt + jnp.where(valid, 1.0, 0.0)
            vscr[...] = jnp.where(valid, 1.0, 0.0)
            for k in range(4):
                mv = plsc.load_gather(vscr, [box_sel[k]])
                off = row0 * 4 + k * _L
                box_st[pl.ds(off, _L)] = jnp.where(
                    mv > 0.0, bbuf[pl.ds(off, _L)], 0.0)
            return cnt

        cnt = lax.fori_loop(0, _GROUPS, grp_body, cnt)

        @pl.when(g + 2 < _NCHUNK)
        def _():
            start_chunk_dma(g + 2, buf, sem)

        return cnt

    def pair_body(p, cnt):
        g = p * 2
        cnt = chunk_step(g, 0, cnt)
        cnt = chunk_step(g + 1, 1, cnt)
        return cnt

    cnt = lax.fori_loop(0, _NCHUNK // 2, pair_body,
                        jnp.zeros((_L,), jnp.float32))

    vscr[...] = cnt
    pltpu.sync_copy(vscr, counts_hbm.at[wid])
    pltpu.sync_copy(lab_st, labels_hbm.at[pl.ds(rbase, _RPW)])
    pltpu.sync_copy(box_st, boxes_out_hbm.at[pl.ds(rbase * 4, _RPW * 4)])


def _finalize_body(cref, oref):
    oref[...] = lax.broadcast(jnp.maximum(jnp.sum(cref[...]), 1.0), (1, 1))


def kernel(pred_logits, pred_boxes):
    boxes_flat = pred_boxes.reshape(_R * 4)
    mesh = plsc.VectorSubcoreMesh(core_axis_name="c", subcore_axis_name="s")
    labels_flat, boxes_out, counts = pl.kernel(
        _sc_body,
        out_type=(
            jax.ShapeDtypeStruct((_R,), jnp.int32),
            jax.ShapeDtypeStruct((_R * 4,), jnp.float32),
            jax.ShapeDtypeStruct((_NW, _L), jnp.float32),
        ),
        mesh=mesh,
        compiler_params=pltpu.CompilerParams(needs_layout_passes=False),
        scratch_types=[
            pltpu.VMEM((_CHUNK, _C), jnp.float32),
            pltpu.VMEM((_CHUNK, _C), jnp.float32),
            pltpu.VMEM((_RPW * 4,), jnp.float32),
            pltpu.VMEM((_RPW,), jnp.int32),
            pltpu.VMEM((_RPW * 4,), jnp.float32),
            pltpu.VMEM((_L,), jnp.float32),
            pltpu.SemaphoreType.DMA,
            pltpu.SemaphoreType.DMA,
            pltpu.SemaphoreType.DMA,
        ],
    )(pred_logits, boxes_flat)
    num_boxes = pl.pallas_call(
        _finalize_body,
        out_shape=jax.ShapeDtypeStruct((1, 1), jnp.float32),
    )(counts)[0, 0]
    return (labels_flat.reshape(_B, _Q),
            boxes_out.reshape(_B, _Q, 4),
            num_boxes)


# speedup vs baseline: 2.1946x; 1.0646x over previous
import jax
import jax.numpy as jnp
from jax import lax
from jax.experimental import pallas as pl
from jax.experimental.pallas import tpu as pltpu
from jax.experimental.pallas import tpu_sc as plsc

_B, _Q, _C = 64, 2048, 256
_R = _B * _Q
_NC, _NS, _L = 2, 16, 16
_NW = _NC * _NS
_RPW = _R // _NW
_BPW = _RPW // _Q
_CHUNK = 128
_NCHUNK = _RPW // _CHUNK
_GROUPS = _CHUNK // _L
_BLK = 8
_MAIN_C = 240


def _sc_body(logits_hbm, labels_hbm, lbuf0, lbuf1, lab_st, sem0, sem1):
    cid = lax.axis_index("c")
    sid = lax.axis_index("s")
    wid = sid * _NC + cid
    b0 = wid * _BPW

    lane = lax.iota(jnp.int32, _L)
    neg_inf = jnp.full((_L,), -jnp.inf, jnp.float32)

    lbufs = (lbuf0, lbuf1)
    sems = (sem0, sem1)

    def start_chunk_dma(g, buf, sem):
        bb = b0 + jnp.right_shift(g, 4)
        q0 = jnp.bitwise_and(g, 15) * _CHUNK
        pltpu.async_copy(logits_hbm.at[bb, pl.ds(q0, _CHUNK), :], buf, sem)

    start_chunk_dma(jnp.int32(0), lbuf0, sem0)
    start_chunk_dma(jnp.int32(1), lbuf1, sem1)

    def chunk_step(g, b):
        buf = lbufs[b]
        sem = sems[b]
        pltpu.make_async_copy(
            logits_hbm.at[0, pl.ds(0, _CHUNK), :], buf, sem).wait()

        def grp_body(grp, _):
            row0 = g * _CHUNK + grp * _L
            rows = lane + grp * _L

            def blk_body(blk, carry):
                best, bblk = carry
                c = blk * _BLK
                vs = []
                col = lane + c
                for j in range(_BLK):
                    if j:
                        col = col + 1
                    vs.append(plsc.load_gather(buf, [rows, col]))
                m01 = jnp.maximum(vs[0], vs[1])
                m23 = jnp.maximum(vs[2], vs[3])
                m45 = jnp.maximum(vs[4], vs[5])
                m67 = jnp.maximum(vs[6], vs[7])
                m = jnp.maximum(jnp.maximum(m01, m23),
                                jnp.maximum(m45, m67))
                gt = m > best
                best = jnp.where(gt, m, best)
                bblk = jnp.where(gt, jnp.full((_L,), c, jnp.int32), bblk)
                return (best, bblk)

            best, bblk = lax.fori_loop(
                0, _MAIN_C // _BLK, blk_body,
                (neg_inf, jnp.zeros((_L,), jnp.int32)))

            col = bblk + lane
            v = plsc.load_gather(buf, [rows, col])
            bcol = col
            found = v == best
            for _ in range(_BLK - 1):
                col = col + 1
                v = plsc.load_gather(buf, [rows, col])
                hit = jnp.logical_and(v == best,
                                      jnp.logical_not(found))
                bcol = jnp.where(hit, col, bcol)
                found = jnp.logical_or(found, hit)

            def tail_body(_, carry):
                best, bcol, col = carry
                col = jnp.bitwise_and(col + 1, _C - 1)
                v = plsc.load_gather(buf, [rows, col])
                gt = v > best
                best = jnp.where(gt, v, best)
                bcol = jnp.where(gt, col, bcol)
                return (best, bcol, col)

            best, bcol, _ = lax.fori_loop(
                _MAIN_C, _C, tail_body,
                (best, bcol, lane + (_MAIN_C - 1)))

            lab_st[pl.ds(row0, _L)] = jnp.where(best > 0.0, bcol, -1)
            return 0

        lax.fori_loop(0, _GROUPS, grp_body, 0)

        @pl.when(g + 2 < _NCHUNK)
        def _():
            start_chunk_dma(g + 2, buf, sem)

    def pair_body(p, _):
        g = p * 2
        chunk_step(g, 0)
        chunk_step(g + 1, 1)
        return 0

    lax.fori_loop(0, _NCHUNK // 2, pair_body, 0)

    for i in range(_BPW):
        pltpu.sync_copy(lab_st.at[pl.ds(i * _Q, _Q)], labels_hbm.at[b0 + i])


_TCB = 8


def _boxes_body(lab_ref, box_ref, out_ref, cnt_ref):
    step = pl.program_id(0)
    valid = lab_ref[...] >= 0
    for comp in range(4):
        out_ref[:, :, comp] = jnp.where(valid, box_ref[:, :, comp], 0.0)
    c = jnp.sum(valid.astype(jnp.float32))

    @pl.when(step == 0)
    def _():
        cnt_ref[...] = jnp.zeros_like(cnt_ref)

    cnt_ref[...] += lax.broadcast(c, (1, 1))

    @pl.when(step == (_B // _TCB) - 1)
    def _():
        cnt_ref[...] = jnp.maximum(cnt_ref[...], 1.0)


def kernel(pred_logits, pred_boxes):
    mesh = plsc.VectorSubcoreMesh(core_axis_name="c", subcore_axis_name="s")
    labels = pl.kernel(
        _sc_body,
        out_type=jax.ShapeDtypeStruct((_B, _Q), jnp.int32),
        mesh=mesh,
        compiler_params=pltpu.CompilerParams(needs_layout_passes=False),
        scratch_types=[
            pltpu.VMEM((_CHUNK, _C), jnp.float32),
            pltpu.VMEM((_CHUNK, _C), jnp.float32),
            pltpu.VMEM((_RPW,), jnp.int32),
            pltpu.SemaphoreType.DMA,
            pltpu.SemaphoreType.DMA,
        ],
    )(pred_logits)
    boxes_out, num_boxes = pl.pallas_call(
        _boxes_body,
        grid=(_B // _TCB,),
        in_specs=[
            pl.BlockSpec((_TCB, _Q), lambda i: (i, 0)),
            pl.BlockSpec((_TCB, _Q, 4), lambda i: (i, 0, 0)),
        ],
        out_specs=[
            pl.BlockSpec((_TCB, _Q, 4), lambda i: (i, 0, 0)),
            pl.BlockSpec((1, 1), lambda i: (0, 0)),
        ],
        out_shape=[
            jax.ShapeDtypeStruct((_B, _Q, 4), jnp.float32),
            jax.ShapeDtypeStruct((1, 1), jnp.float32),
        ],
    )(labels, pred_boxes)
    return labels, boxes_out, num_boxes[0, 0]


# speedup vs baseline: 4.6483x; 2.1181x over previous
import jax
import jax.numpy as jnp
from jax import lax
from jax.experimental import pallas as pl
from jax.experimental.pallas import tpu as pltpu
from jax.experimental.pallas import tpu_sc as plsc

_B, _Q, _C = 64, 2048, 256
_R = _B * _Q
_NC, _NS, _L = 2, 16, 16
_NW = _NC * _NS
_RPW = _R // _NW
_BPW = _RPW // _Q
_CHUNK = 128
_NCHUNK = _RPW // _CHUNK
_GROUPS = _CHUNK // _L
_BLK = 8
_MAIN_C = 240


def _sc_body(logits_hbm, boxes_hbm, labels_hbm, boxes_out_hbm, counts_hbm,
             lbuf0, lbuf1, bbuf, lab_st, box_st, vscr, sem_b, sem0, sem1):
    cid = lax.axis_index("c")
    sid = lax.axis_index("s")
    wid = sid * _NC + cid
    b0 = wid * _BPW

    lane = lax.iota(jnp.int32, _L)
    lane_d4 = jnp.right_shift(lane, 2)
    box_sel = [lane_d4 + (4 * k) for k in range(4)]
    neg_inf = jnp.full((_L,), -jnp.inf, jnp.float32)

    lbufs = (lbuf0, lbuf1)
    sems = (sem0, sem1)

    def start_chunk_dma(g, buf, sem):
        bb = b0 + jnp.right_shift(g, 4)
        q0 = jnp.bitwise_and(g, 15) * _CHUNK
        pltpu.async_copy(logits_hbm.at[bb, pl.ds(q0, _CHUNK), :], buf, sem)

    for i in range(_BPW):
        cp = pltpu.async_copy(
            boxes_hbm.at[b0 + i], bbuf.at[pl.ds(i * _Q * 4, _Q * 4)], sem_b)
    start_chunk_dma(jnp.int32(0), lbuf0, sem0)
    start_chunk_dma(jnp.int32(1), lbuf1, sem1)
    cp.wait()
    cp.wait()

    def chunk_step(g, b, cnt):
        buf = lbufs[b]
        sem = sems[b]
        pltpu.make_async_copy(
            logits_hbm.at[0, pl.ds(0, _CHUNK), :], buf, sem).wait()

        def grp_body(grp, cnt):
            row0 = g * _CHUNK + grp * _L
            rows = lane + grp * _L

            def blk_body(blk, carry):
                best, bblk = carry
                c = blk * _BLK
                vs = []
                col = lane + c
                for j in range(_BLK):
                    if j:
                        col = col + 1
                    vs.append(plsc.load_gather(buf, [rows, col]))
                m01 = jnp.maximum(vs[0], vs[1])
                m23 = jnp.maximum(vs[2], vs[3])
                m45 = jnp.maximum(vs[4], vs[5])
                m67 = jnp.maximum(vs[6], vs[7])
                m = jnp.maximum(jnp.maximum(m01, m23),
                                jnp.maximum(m45, m67))
                gt = m > best
                best = jnp.where(gt, m, best)
                bblk = jnp.where(gt, jnp.full((_L,), c, jnp.int32), bblk)
                return (best, bblk)

            best, bblk = lax.fori_loop(
                0, _MAIN_C // _BLK, blk_body,
                (neg_inf, jnp.zeros((_L,), jnp.int32)))

            col = bblk + lane
            v = plsc.load_gather(buf, [rows, col])
            bcol = col
            found = v == best
            for _ in range(_BLK - 1):
                col = col + 1
                v = plsc.load_gather(buf, [rows, col])
                hit = jnp.logical_and(v == best,
                                      jnp.logical_not(found))
                bcol = jnp.where(hit, col, bcol)
                found = jnp.logical_or(found, hit)

            def tail_body(_, carry):
                best, bcol, col = carry
                col = jnp.bitwise_and(col + 1, _C - 1)
                v = plsc.load_gather(buf, [rows, col])
                gt = v > best
                best = jnp.where(gt, v, best)
                bcol = jnp.where(gt, col, bcol)
                return (best, bcol, col)

            best, bcol, _ = lax.fori_loop(
                _MAIN_C, _C, tail_body,
                (best, bcol, lane + (_MAIN_C - 1)))

            valid = best > 0.0
            lab_st[pl.ds(row0, _L)] = jnp.where(valid, bcol, -1)
            cnt = cnt + jnp.where(valid, 1.0, 0.0)
            vscr[...] = jnp.where(valid, 1.0, 0.0)
            off = row0 * 4
            for k in range(4):
                mv = plsc.load_gather(vscr, [box_sel[k]])
                bx = bbuf[pl.ds(off + k * _L, _L)]
                box_st[pl.ds(off + k * _L, _L)] = jnp.where(
                    mv > 0.0, bx, 0.0)
            return cnt

        cnt = lax.fori_loop(0, _GROUPS, grp_body, cnt)

        @pl.when(g + 2 < _NCHUNK)
        def _():
            start_chunk_dma(g + 2, buf, sem)

        return cnt

    def pair_body(p, cnt):
        g = p * 2
        cnt = chunk_step(g, 0, cnt)
        cnt = chunk_step(g + 1, 1, cnt)
        return cnt

    cnt = lax.fori_loop(0, _NCHUNK // 2, pair_body,
                        jnp.zeros((_L,), jnp.float32))

    vscr[...] = cnt
    pltpu.sync_copy(vscr, counts_hbm.at[wid])
    for i in range(_BPW):
        pltpu.sync_copy(lab_st.at[pl.ds(i * _Q, _Q)], labels_hbm.at[b0 + i])
        pltpu.sync_copy(box_st.at[pl.ds(i * _Q * 4, _Q * 4)],
                        boxes_out_hbm.at[b0 + i])


def _finalize_body(cref, oref):
    oref[...] = lax.broadcast(jnp.maximum(jnp.sum(cref[...]), 1.0), (1, 1))


def kernel(pred_logits, pred_boxes):
    boxes2d = pred_boxes.reshape(_B, _Q * 4)
    mesh = plsc.VectorSubcoreMesh(core_axis_name="c", subcore_axis_name="s")
    labels, boxes_out2d, counts = pl.kernel(
        _sc_body,
        out_type=(
            jax.ShapeDtypeStruct((_B, _Q), jnp.int32),
            jax.ShapeDtypeStruct((_B, _Q * 4), jnp.float32),
            jax.ShapeDtypeStruct((_NW, _L), jnp.float32),
        ),
        mesh=mesh,
        compiler_params=pltpu.CompilerParams(needs_layout_passes=False),
        scratch_types=[
            pltpu.VMEM((_CHUNK, _C), jnp.float32),
            pltpu.VMEM((_CHUNK, _C), jnp.float32),
            pltpu.VMEM((_RPW * 4,), jnp.float32),
            pltpu.VMEM((_RPW,), jnp.int32),
            pltpu.VMEM((_RPW * 4,), jnp.float32),
            pltpu.VMEM((_L,), jnp.float32),
            pltpu.SemaphoreType.DMA,
            pltpu.SemaphoreType.DMA,
            pltpu.SemaphoreType.DMA,
        ],
    )(pred_logits, boxes2d)
    num_boxes = pl.pallas_call(
        _finalize_body,
        out_shape=jax.ShapeDtypeStruct((1, 1), jnp.float32),
    )(counts)[0, 0]
    return labels, boxes_out2d.reshape(_B, _Q, 4), num_boxes
